# baseline (speedup 1.0000x reference)
import jax
import jax.numpy as jnp
from jax import lax
from jax.experimental import pallas as pl
from jax.experimental.pallas import tpu as pltpu

_PAD = 1
_B_ROWS = 4
_SEQ = 2048
_NUM_IDX = _B_ROWS * _SEQ
_DIM = 1024
_ROWS_PER_STEP = 128
_STEPS = _NUM_IDX // _ROWS_PER_STEP
_TAB_ROWS = 2050


def _positions_body(tok_ref, pos_ref):
    tok = tok_ref[...]
    mask = (tok != _PAD).astype(jnp.int32)
    col = lax.broadcasted_iota(jnp.int32, (_B_ROWS, _SEQ), 1)
    csum = mask
    shift = 1
    while shift < _SEQ:
        csum = csum + jnp.where(col >= shift, jnp.roll(csum, shift, axis=1), 0)
        shift *= 2
    pos_ref[...] = csum * mask + _PAD


def _tc_gather_body(idx_ref, table_ref, out_ref):
    i = pl.program_id(0)
    for j in range(_ROWS_PER_STEP):
        r = idx_ref[i * _ROWS_PER_STEP + j]
        out_ref[pl.ds(j, 1)] = table_ref[pl.ds(r, 1)]


def kernel(tokens, emb_table):
    tokens = tokens.astype(jnp.int32)
    positions = pl.pallas_call(
        _positions_body,
        out_shape=jax.ShapeDtypeStruct((_B_ROWS, _SEQ), jnp.int32),
    )(tokens)

    table3 = emb_table.reshape(_TAB_ROWS, 8, 128)
    grid_spec = pltpu.PrefetchScalarGridSpec(
        num_scalar_prefetch=1,
        grid=(_STEPS,),
        in_specs=[
            pl.BlockSpec((_TAB_ROWS, 8, 128), lambda i, idx: (0, 0, 0)),
        ],
        out_specs=pl.BlockSpec((_ROWS_PER_STEP, 8, 128), lambda i, idx: (i, 0, 0)),
    )
    out = pl.pallas_call(
        _tc_gather_body,
        grid_spec=grid_spec,
        out_shape=jax.ShapeDtypeStruct((_NUM_IDX, 8, 128), jnp.float32),
        compiler_params=pltpu.CompilerParams(
            dimension_semantics=("arbitrary",),
        ),
    )(positions.reshape(_NUM_IDX), table3)
    return out.reshape(_B_ROWS, _SEQ, _DIM)


# speedup vs baseline: 1.0003x; 1.0003x over previous
import jax
import jax.numpy as jnp
from jax import lax
from jax.experimental import pallas as pl
from jax.experimental.pallas import tpu as pltpu

_PAD = 1
_B_ROWS = 4
_SEQ = 2048
_NUM_IDX = _B_ROWS * _SEQ
_DIM = 1024
_ROWS_PER_STEP = 128
_STEPS = _NUM_IDX // _ROWS_PER_STEP
_TAB_ROWS = 2050


def _positions_body(tok_ref, pos_ref):
    tok = tok_ref[...]
    mask = (tok != _PAD).astype(jnp.int32)
    col = lax.broadcasted_iota(jnp.int32, (_B_ROWS, _SEQ), 1)
    csum = mask
    shift = 1
    while shift < _SEQ:
        csum = csum + jnp.where(col >= shift, jnp.roll(csum, shift, axis=1), 0)
        shift *= 2
    pos_ref[...] = csum * mask + _PAD


def _tc_gather_body(idx_ref, table_hbm, out_ref, table_vmem, sem):
    i = pl.program_id(0)

    @pl.when(i == 0)
    def _():
        copy = pltpu.make_async_copy(table_hbm, table_vmem, sem)
        copy.start()
        copy.wait()

    for j in range(_ROWS_PER_STEP):
        r = idx_ref[i * _ROWS_PER_STEP + j]
        out_ref[pl.ds(j, 1)] = table_vmem[pl.ds(r, 1)]


def kernel(tokens, emb_table):
    tokens = tokens.astype(jnp.int32)
    positions = pl.pallas_call(
        _positions_body,
        out_shape=jax.ShapeDtypeStruct((_B_ROWS, _SEQ), jnp.int32),
    )(tokens)

    table3 = emb_table.reshape(_TAB_ROWS, 8, 128)
    grid_spec = pltpu.PrefetchScalarGridSpec(
        num_scalar_prefetch=1,
        grid=(_STEPS,),
        in_specs=[
            pl.BlockSpec(memory_space=pltpu.HBM),
        ],
        out_specs=pl.BlockSpec((_ROWS_PER_STEP, 8, 128), lambda i, idx: (i, 0, 0)),
        scratch_shapes=[
            pltpu.VMEM((_TAB_ROWS, 8, 128), jnp.float32),
            pltpu.SemaphoreType.DMA,
        ],
    )
    out = pl.pallas_call(
        _tc_gather_body,
        grid_spec=grid_spec,
        out_shape=jax.ShapeDtypeStruct((_NUM_IDX, 8, 128), jnp.float32),
        compiler_params=pltpu.CompilerParams(
            dimension_semantics=("arbitrary",),
        ),
    )(positions.reshape(_NUM_IDX), table3)
    return out.reshape(_B_ROWS, _SEQ, _DIM)
